# Initial kernel scaffold; baseline (speedup 1.0000x reference)
import jax
import jax.numpy as jnp
from jax.experimental import pallas as pl


def kernel(x):
    raise NotImplementedError("write your pallas kernel here")


# speedup vs baseline: 694.6321x; 694.6321x over previous
import math

import jax
import jax.numpy as jnp
import numpy as np
from jax.experimental import pallas as pl

N = 65536
N1 = 256
N2 = 256

_nk = np.outer(np.arange(N1, dtype=np.float64), np.arange(N2, dtype=np.float64))
_Fr = np.cos(2.0 * np.pi * _nk / N1).astype(np.float32)
_Fi = (-np.sin(2.0 * np.pi * _nk / N1)).astype(np.float32)
_Tr = np.cos(2.0 * np.pi * _nk / N).astype(np.float32)
_Ti = (-np.sin(2.0 * np.pi * _nk / N)).astype(np.float32)


def _fft_kernel(ar_ref, ai_ref, fr_ref, fi_ref, tr_ref, ti_ref,
                dr_ref, di_ref):
    ar = ar_ref[...]
    ai = ai_ref[...]
    fr = fr_ref[...]
    fi = fi_ref[...]

    dot = lambda a, b: jax.lax.dot(a, b, precision=jax.lax.Precision.HIGHEST,
                                   preferred_element_type=jnp.float32)

    br = dot(ar, fr) - dot(ai, fi)
    bi = dot(ar, fi) + dot(ai, fr)

    tr = tr_ref[...]
    ti = ti_ref[...]
    cr = tr * br - ti * bi
    ci = tr * bi + ti * br

    dr_ref[...] = dot(fr, cr) - dot(fi, ci)
    di_ref[...] = dot(fr, ci) + dot(fi, cr)


def kernel(x):
    ar = x[:, 0].reshape(N2, N1).T
    ai = x[:, 1].reshape(N2, N1).T

    fr = jnp.asarray(_Fr)
    fi = jnp.asarray(_Fi)
    tr = jnp.asarray(_Tr)
    ti = jnp.asarray(_Ti)

    dr, di = pl.pallas_call(
        _fft_kernel,
        out_shape=(
            jax.ShapeDtypeStruct((N1, N2), jnp.float32),
            jax.ShapeDtypeStruct((N1, N2), jnp.float32),
        ),
    )(ar, ai, fr, fi, tr, ti)

    return jnp.stack((dr.reshape(-1), di.reshape(-1)), axis=-1)
